# Initial kernel scaffold; baseline (speedup 1.0000x reference)
import jax
import jax.numpy as jnp
from jax.experimental import pallas as pl


def kernel(x, edge_index, edge_attr, graph_ids, Wn, bn, We, be, W1, b1, W2, b2, Wl, bl):
    raise NotImplementedError("write your pallas kernel here")


# speedup vs baseline: 3.2187x; 3.2187x over previous
import functools

import jax
import jax.numpy as jnp
from jax import lax
from jax.experimental import pallas as pl
from jax.experimental.pallas import tpu as pltpu
from jax.experimental.pallas import tpu_sc as plsc

N_NODES = 10000
N_EDGES = 320000
DIM = 128
NUM_GRAPHS = 256
T = 4

NC = 2
NS = 16
NW = NC * NS
EDGES_PER_TILE = N_EDGES // NW
CHUNK = 80
NCHUNK = EDGES_PER_TILE // CHUNK
N_PAD = 10240
ROWS_PER_TILE = N_PAD // NS


def _linear_body(x_ref, w_ref, b_ref, o_ref):
    o_ref[...] = (
        jnp.dot(x_ref[...], w_ref[...], preferred_element_type=jnp.float32)
        + b_ref[...]
    )


def _linear(x, W, b, block_rows):
    rows, k = x.shape
    n = W.shape[1]
    grid = rows // block_rows
    return pl.pallas_call(
        _linear_body,
        grid=(grid,),
        in_specs=[
            pl.BlockSpec((block_rows, k), lambda i: (i, 0)),
            pl.BlockSpec((k, n), lambda i: (0, 0)),
            pl.BlockSpec((1, n), lambda i: (0, 0)),
        ],
        out_specs=pl.BlockSpec((block_rows, n), lambda i: (i, 0)),
        out_shape=jax.ShapeDtypeStruct((rows, n), jnp.float32),
    )(x, W, b.reshape(1, n))


def _mlp_body(h_ref, a0_ref, a1_ref, w1_ref, b1_ref, w2_ref, b2_ref, o_ref):
    z = h_ref[...] + a0_ref[0] + a1_ref[0]
    u = jnp.maximum(
        jnp.dot(z, w1_ref[...], preferred_element_type=jnp.float32) + b1_ref[...],
        0.0,
    )
    o_ref[...] = (
        jnp.dot(u, w2_ref[...], preferred_element_type=jnp.float32) + b2_ref[...]
    )


def _mlp(h, ag, W1, b1, W2, b2):
    br = 1000
    grid = N_NODES // br
    return pl.pallas_call(
        _mlp_body,
        grid=(grid,),
        in_specs=[
            pl.BlockSpec((br, DIM), lambda i: (i, 0)),
            pl.BlockSpec((1, br, DIM), lambda i: (0, i, 0)),
            pl.BlockSpec((1, br, DIM), lambda i: (1, i, 0)),
            pl.BlockSpec((DIM, DIM), lambda i: (0, 0)),
            pl.BlockSpec((1, DIM), lambda i: (0, 0)),
            pl.BlockSpec((DIM, DIM), lambda i: (0, 0)),
            pl.BlockSpec((1, DIM), lambda i: (0, 0)),
        ],
        out_specs=pl.BlockSpec((br, DIM), lambda i: (i, 0)),
        out_shape=jax.ShapeDtypeStruct((N_NODES, DIM), jnp.float32),
    )(h, ag, ag, W1, b1.reshape(1, DIM), W2, b2.reshape(1, DIM))


def _final_body(h0, h1, h2, h3, wl_ref, g_ref, bl_ref, o_ref):
    i = pl.program_id(0)
    hs = (h0, h1, h2, h3)
    br = h0.shape[0]
    acc = jnp.zeros((br, DIM), jnp.float32)
    for t in range(T):
        acc = acc + jnp.dot(
            hs[t][...],
            wl_ref[t * DIM : (t + 1) * DIM, :],
            preferred_element_type=jnp.float32,
        )
    ids = g_ref[0, 0, :]
    onehot = (
        lax.broadcasted_iota(jnp.int32, (br, NUM_GRAPHS), 1) == ids[:, None]
    ).astype(jnp.float32)
    pooled = lax.dot_general(
        onehot, acc, (((0,), (0,)), ((), ())), preferred_element_type=jnp.float32
    )

    @pl.when(i == 0)
    def _():
        o_ref[...] = jnp.broadcast_to(bl_ref[...], (NUM_GRAPHS, DIM))

    o_ref[...] += pooled


def _final(hs, Wl, graph_ids, bl):
    br = 1000
    grid = N_NODES // br
    g3 = graph_ids.reshape(grid, 1, br)
    return pl.pallas_call(
        _final_body,
        grid=(grid,),
        in_specs=[
            pl.BlockSpec((br, DIM), lambda i: (i, 0)),
            pl.BlockSpec((br, DIM), lambda i: (i, 0)),
            pl.BlockSpec((br, DIM), lambda i: (i, 0)),
            pl.BlockSpec((br, DIM), lambda i: (i, 0)),
            pl.BlockSpec((T * DIM, DIM), lambda i: (0, 0)),
            pl.BlockSpec((1, 1, br), lambda i: (i, 0, 0)),
            pl.BlockSpec((1, DIM), lambda i: (0, 0)),
        ],
        out_specs=pl.BlockSpec((NUM_GRAPHS, DIM), lambda i: (0, 0)),
        out_shape=jax.ShapeDtypeStruct((NUM_GRAPHS, DIM), jnp.float32),
    )(hs[0], hs[1], hs[2], hs[3], Wl, g3, bl.reshape(1, DIM))


def _edge_pass(h, ea3, src3, dst3):
    mesh = plsc.VectorSubcoreMesh(core_axis_name="c", subcore_axis_name="s")

    @functools.partial(
        pl.kernel,
        mesh=mesh,
        out_type=jax.ShapeDtypeStruct((NC, N_PAD, DIM), jnp.float32),
        scratch_types=[
            pltpu.VMEM((1, CHUNK), jnp.int32),
            pltpu.VMEM((1, CHUNK), jnp.int32),
            pltpu.VMEM((CHUNK, DIM), jnp.float32),
            pltpu.VMEM((CHUNK, DIM), jnp.float32),
            pltpu.VMEM_SHARED((N_PAD, DIM), jnp.float32),
            pltpu.SemaphoreType.DMA,
        ],
    )
    def k(h_hbm, ea_hbm, src_hbm, dst_hbm, out_hbm, srcv, dstv, eav, gv, aggr, sem):
        c = lax.axis_index("c")
        s = lax.axis_index("s")
        wid = s * NC + c

        zero = jnp.zeros((16,), jnp.float32)

        def zero_body(r, _):
            for q in range(DIM // 16):
                gv[r, pl.ds(q * 16, 16)] = zero
            return 0

        lax.fori_loop(0, CHUNK, zero_body, 0)
        row0 = s * ROWS_PER_TILE
        for z in range(ROWS_PER_TILE // CHUNK):
            pltpu.sync_copy(gv, aggr.at[pl.ds(row0 + z * CHUNK, CHUNK)])
        plsc.subcore_barrier()

        def chunk_body(j, _):
            cid = wid * NCHUNK + j
            pltpu.sync_copy(src_hbm.at[cid], srcv)
            pltpu.sync_copy(dst_hbm.at[cid], dstv)
            gather = pltpu.async_copy(h_hbm.at[srcv.at[0]], gv, sem)
            pltpu.sync_copy(ea_hbm.at[cid], eav)
            gather.wait()

            def edge_body(e, _):
                for q in range(DIM // 16):
                    sl = pl.ds(q * 16, 16)
                    gv[e, sl] = jnp.maximum(gv[e, sl] + eav[e, sl], 0.0)
                return 0

            lax.fori_loop(0, CHUNK, edge_body, 0)
            pltpu.sync_copy(gv, aggr.at[dstv.at[0]], add=True)
            return 0

        lax.fori_loop(0, NCHUNK, chunk_body, 0)
        plsc.subcore_barrier()

        for z in range(ROWS_PER_TILE // CHUNK):
            r0 = row0 + z * CHUNK
            pltpu.sync_copy(aggr.at[pl.ds(r0, CHUNK)], out_hbm.at[c, pl.ds(r0, CHUNK)])

    return k(h, ea3, src3, dst3)


def kernel(x, edge_index, edge_attr, graph_ids, Wn, bn, We, be, W1, b1, W2, b2, Wl, bl):
    h = _linear(x, Wn, bn, 1000)
    ea = _linear(edge_attr, We, be, 4000)
    ea3 = ea.reshape(NW * NCHUNK, CHUNK, DIM)
    src3 = edge_index[0].reshape(NW * NCHUNK, 1, CHUNK)
    dst3 = edge_index[1].reshape(NW * NCHUNK, 1, CHUNK)
    hs = []
    for _ in range(T):
        ag = _edge_pass(h, ea3, src3, dst3)
        h = _mlp(h, ag, W1, b1, W2, b2)
        hs.append(h)
    return _final(hs, Wl, graph_ids, bl)


# speedup vs baseline: 6.3390x; 1.9694x over previous
import functools

import jax
import jax.numpy as jnp
from jax import lax
from jax.experimental import pallas as pl
from jax.experimental.pallas import tpu as pltpu
from jax.experimental.pallas import tpu_sc as plsc

N_NODES = 10000
N_EDGES = 320000
DIM = 128
NUM_GRAPHS = 256
T = 4

NC = 2
NS = 16
NW = NC * NS
EDGES_PER_TILE = N_EDGES // NW
CHUNK = 80
NCHUNK = EDGES_PER_TILE // CHUNK
N_PAD = 10240
ROWS_PER_TILE = N_PAD // NS


def _linear_body(x_ref, w_ref, b_ref, o_ref):
    o_ref[...] = (
        jnp.dot(x_ref[...], w_ref[...], preferred_element_type=jnp.float32)
        + b_ref[...]
    )


def _linear(x, W, b, block_rows):
    rows, k = x.shape
    n = W.shape[1]
    grid = rows // block_rows
    return pl.pallas_call(
        _linear_body,
        grid=(grid,),
        in_specs=[
            pl.BlockSpec((block_rows, k), lambda i: (i, 0)),
            pl.BlockSpec((k, n), lambda i: (0, 0)),
            pl.BlockSpec((1, n), lambda i: (0, 0)),
        ],
        out_specs=pl.BlockSpec((block_rows, n), lambda i: (i, 0)),
        out_shape=jax.ShapeDtypeStruct((rows, n), jnp.float32),
    )(x, W, b.reshape(1, n))


def _mlp_body(h_ref, a0_ref, a1_ref, w1_ref, b1_ref, w2_ref, b2_ref, o_ref):
    z = h_ref[...] + a0_ref[0] + a1_ref[0]
    u = jnp.maximum(
        jnp.dot(z, w1_ref[...], preferred_element_type=jnp.float32) + b1_ref[...],
        0.0,
    )
    o_ref[...] = (
        jnp.dot(u, w2_ref[...], preferred_element_type=jnp.float32) + b2_ref[...]
    )


def _mlp(h, ag, W1, b1, W2, b2):
    br = 1000
    grid = N_NODES // br
    return pl.pallas_call(
        _mlp_body,
        grid=(grid,),
        in_specs=[
            pl.BlockSpec((br, DIM), lambda i: (i, 0)),
            pl.BlockSpec((1, br, DIM), lambda i: (0, i, 0)),
            pl.BlockSpec((1, br, DIM), lambda i: (1, i, 0)),
            pl.BlockSpec((DIM, DIM), lambda i: (0, 0)),
            pl.BlockSpec((1, DIM), lambda i: (0, 0)),
            pl.BlockSpec((DIM, DIM), lambda i: (0, 0)),
            pl.BlockSpec((1, DIM), lambda i: (0, 0)),
        ],
        out_specs=pl.BlockSpec((br, DIM), lambda i: (i, 0)),
        out_shape=jax.ShapeDtypeStruct((N_NODES, DIM), jnp.float32),
    )(h, ag, ag, W1, b1.reshape(1, DIM), W2, b2.reshape(1, DIM))


def _final_body(h0, h1, h2, h3, wl_ref, g_ref, bl_ref, o_ref):
    i = pl.program_id(0)
    hs = (h0, h1, h2, h3)
    br = h0.shape[0]
    acc = jnp.zeros((br, DIM), jnp.float32)
    for t in range(T):
        acc = acc + jnp.dot(
            hs[t][...],
            wl_ref[t * DIM : (t + 1) * DIM, :],
            preferred_element_type=jnp.float32,
        )
    ids = g_ref[0, 0, :]
    onehot = (
        lax.broadcasted_iota(jnp.int32, (br, NUM_GRAPHS), 1) == ids[:, None]
    ).astype(jnp.float32)
    pooled = lax.dot_general(
        onehot, acc, (((0,), (0,)), ((), ())), preferred_element_type=jnp.float32
    )

    @pl.when(i == 0)
    def _():
        o_ref[...] = jnp.broadcast_to(bl_ref[...], (NUM_GRAPHS, DIM))

    o_ref[...] += pooled


def _final(hs, Wl, graph_ids, bl):
    br = 1000
    grid = N_NODES // br
    g3 = graph_ids.reshape(grid, 1, br)
    return pl.pallas_call(
        _final_body,
        grid=(grid,),
        in_specs=[
            pl.BlockSpec((br, DIM), lambda i: (i, 0)),
            pl.BlockSpec((br, DIM), lambda i: (i, 0)),
            pl.BlockSpec((br, DIM), lambda i: (i, 0)),
            pl.BlockSpec((br, DIM), lambda i: (i, 0)),
            pl.BlockSpec((T * DIM, DIM), lambda i: (0, 0)),
            pl.BlockSpec((1, 1, br), lambda i: (i, 0, 0)),
            pl.BlockSpec((1, DIM), lambda i: (0, 0)),
        ],
        out_specs=pl.BlockSpec((NUM_GRAPHS, DIM), lambda i: (0, 0)),
        out_shape=jax.ShapeDtypeStruct((NUM_GRAPHS, DIM), jnp.float32),
    )(hs[0], hs[1], hs[2], hs[3], Wl, g3, bl.reshape(1, DIM))


def _edge_pass(h, ea3, idx3):
    mesh = plsc.VectorSubcoreMesh(core_axis_name="c", subcore_axis_name="s")

    @functools.partial(
        pl.kernel,
        mesh=mesh,
        out_type=jax.ShapeDtypeStruct((NC, N_PAD, DIM), jnp.float32),
        scratch_types=[
            pltpu.VMEM((2, CHUNK), jnp.int32),
            pltpu.VMEM((2, CHUNK), jnp.int32),
            pltpu.VMEM((2, CHUNK), jnp.int32),
            pltpu.VMEM((2, CHUNK), jnp.int32),
            pltpu.VMEM((CHUNK, DIM), jnp.float32),
            pltpu.VMEM((CHUNK, DIM), jnp.float32),
            pltpu.VMEM((CHUNK, DIM), jnp.float32),
            pltpu.VMEM((CHUNK, DIM), jnp.float32),
            pltpu.VMEM_SHARED((N_PAD, DIM), jnp.float32),
            pltpu.SemaphoreType.DMA,
            pltpu.SemaphoreType.DMA,
            pltpu.SemaphoreType.DMA,
            pltpu.SemaphoreType.DMA,
        ],
    )
    def k(h_hbm, ea_hbm, idx_hbm, out_hbm,
          ix0, ix1, ix2, ix3, ea0, ea1, gv0, gv1, aggr, semi, semg, seme, sems):
        c = lax.axis_index("c")
        s = lax.axis_index("s")
        wid = s * NC + c
        ixv = (ix0, ix1, ix2, ix3)
        eav = (ea0, ea1)
        gv = (gv0, gv1)

        zero = jnp.zeros((16,), jnp.float32)

        def zero_body(r, _):
            for q in range(DIM // 16):
                gv0[r, pl.ds(q * 16, 16)] = zero
            return 0

        lax.fori_loop(0, CHUNK, zero_body, 0)
        row0 = s * ROWS_PER_TILE
        for z in range(ROWS_PER_TILE // CHUNK):
            pltpu.sync_copy(gv0, aggr.at[pl.ds(row0 + z * CHUNK, CHUNK)])
        plsc.subcore_barrier()

        def issue_idx(j, r):
            pltpu.async_copy(idx_hbm.at[wid * NCHUNK + j], ixv[r], semi)

        def wait_idx():
            pltpu.make_async_copy(idx_hbm.at[0], ix0, semi).wait()

        def wait_fetch(b):
            pltpu.make_async_copy(h_hbm.at[ix0.at[0]], gv[b], semg).wait()
            pltpu.make_async_copy(ea_hbm.at[0], eav[b], seme).wait()

        def wait_scatter():
            pltpu.make_async_copy(gv0, aggr.at[ix0.at[1]], sems).wait()

        def pwhen(cond, fn):
            if isinstance(cond, bool):
                if cond:
                    fn()
            else:
                pl.when(cond)(fn)

        def slot(j, u):
            b = u % 2

            pwhen(j > 0, wait_scatter)

            pwhen(j + 2 < NCHUNK, lambda: issue_idx(j + 2, (u + 2) % 4))

            def fetch_next():
                wait_idx()
                pltpu.async_copy(
                    h_hbm.at[ixv[(u + 1) % 4].at[0]], gv[1 - b], semg
                )
                pltpu.async_copy(
                    ea_hbm.at[wid * NCHUNK + j + 1], eav[1 - b], seme
                )

            pwhen(j + 1 < NCHUNK, fetch_next)

            wait_fetch(b)
            g = gv[b]
            e_ = eav[b]

            def edge_body(e, _):
                for q in range(DIM // 16):
                    sl = pl.ds(q * 16, 16)
                    g[e, sl] = jnp.maximum(g[e, sl] + e_[e, sl], 0.0)
                return 0

            lax.fori_loop(0, CHUNK, edge_body, 0)
            pltpu.async_copy(g, aggr.at[ixv[u].at[1]], sems, add=True)

        issue_idx(0, 0)
        issue_idx(1, 1)
        wait_idx()
        pltpu.async_copy(h_hbm.at[ix0.at[0]], gv0, semg)
        pltpu.async_copy(ea_hbm.at[wid * NCHUNK], ea0, seme)

        def outer_body(jj, _):
            for u in range(4):
                slot(jj * 4 + u, u)
            return 0

        lax.fori_loop(0, NCHUNK // 4, outer_body, 0)
        for j in range(NCHUNK - NCHUNK % 4, NCHUNK):
            slot(j, j % 4)
        wait_scatter()
        plsc.subcore_barrier()

        for z in range(ROWS_PER_TILE // CHUNK):
            r0 = row0 + z * CHUNK
            pltpu.sync_copy(aggr.at[pl.ds(r0, CHUNK)], out_hbm.at[c, pl.ds(r0, CHUNK)])

    return k(h, ea3, idx3)


def kernel(x, edge_index, edge_attr, graph_ids, Wn, bn, We, be, W1, b1, W2, b2, Wl, bl):
    h = _linear(x, Wn, bn, 1000)
    ea = _linear(edge_attr, We, be, 4000)
    ea3 = ea.reshape(NW * NCHUNK, CHUNK, DIM)
    idx3 = edge_index.reshape(2, NW * NCHUNK, CHUNK).transpose(1, 0, 2)
    hs = []
    for _ in range(T):
        ag = _edge_pass(h, ea3, idx3)
        h = _mlp(h, ag, W1, b1, W2, b2)
        hs.append(h)
    return _final(hs, Wl, graph_ids, bl)
